# baseline (speedup 1.0000x reference)
import functools

import jax
import jax.numpy as jnp
from jax import lax
from jax.experimental import pallas as pl
from jax.experimental.pallas import tpu as pltpu
from jax.experimental.pallas import tpu_sc as plsc

_D = 13
_N = 2 ** _D - 1
_DIM = 128
_A = 16
_OUT = 10

_CHUNK = 512


_LEAVES = 1 << (_D - 1)


def _make_leaf_gather():
    info = plsc.get_sparse_core_info()
    nw = info.num_cores * info.num_subcores
    per_w = _LEAVES // nw
    mesh = plsc.VectorSubcoreMesh(core_axis_name="c", subcore_axis_name="s")

    @functools.partial(
        pl.kernel, mesh=mesh,
        out_type=jax.ShapeDtypeStruct((_LEAVES, _DIM), jnp.float32),
        scratch_types=[
            pltpu.VMEM((per_w,), jnp.int32),
            pltpu.VMEM((per_w, _DIM), jnp.float32),
            pltpu.SemaphoreType.DMA,
        ],
    )
    def leaf_gather(table_hbm, idx_hbm, out_hbm, idx_v, rows_v, sem):
        wid = lax.axis_index("s") * info.num_cores + lax.axis_index("c")
        base = wid * per_w
        pltpu.sync_copy(idx_hbm.at[pl.ds(base, per_w)], idx_v)
        pltpu.async_copy(table_hbm.at[idx_v], rows_v, sem).wait()
        pltpu.sync_copy(rows_v, out_hbm.at[pl.ds(base, per_w)])

    return leaf_gather


_make_leaf_gather = functools.cache(_make_leaf_gather)


def _sweep_kernel(syms_ref, w_ref, b_ref, leaf_b_ref, out_w_ref, out_b_ref,
                  out_ref):
    bv = b_ref[...]
    w_half = (w_ref[...] * 0.5).astype(jnp.bfloat16)
    b_cols = jax.lax.broadcast_in_dim(
        jnp.transpose(bv), (_A, _DIM, _A), (1, 2)
    ).reshape(_A * _DIM, _A).astype(jnp.bfloat16)
    w_aug = jnp.concatenate([w_half, b_cols], axis=1)

    a_iota = jax.lax.broadcasted_iota(jnp.int32, (1, _A), 1)

    def sym_slice(level):
        start = (1 << level) - 1
        return syms_ref[start:start + (1 << level)]

    prev = jnp.tanh(leaf_b_ref[...])

    for level in range(_D - 2, -1, -1):
        length = 1 << level
        c3 = prev.reshape(length, 2, _DIM)
        x2 = c3[:, 0, :] + c3[:, 1, :]
        syms = sym_slice(level)
        s_oh = (syms == a_iota).astype(jnp.bfloat16)
        xa = jnp.concatenate(
            [x2.astype(jnp.bfloat16), s_oh], axis=1)
        sb = jnp.broadcast_to(syms, (length, _DIM))
        chunks = []
        for c0 in range(0, length, _CHUNK):
            c1 = min(c0 + _CHUNK, length)
            y_all = jax.lax.dot_general(
                xa[c0:c1], w_aug, (((1,), (1,)), ((), ())),
                preferred_element_type=jnp.float32)
            sbc = sb[c0:c1]
            acc = jnp.where(sbc == 0, y_all[:, 0:_DIM], 0.0)
            for a in range(1, _A):
                acc = acc + jnp.where(
                    sbc == a, y_all[:, a * _DIM:(a + 1) * _DIM], 0.0)
            chunks.append(acc)
        y = chunks[0] if len(chunks) == 1 else jnp.concatenate(chunks, axis=0)
        prev = jnp.tanh(y)

    out_ref[...] = (
        jax.lax.dot_general(prev, out_w_ref[...], (((1,), (1,)), ((), ())),
                            preferred_element_type=jnp.float32)
        + out_b_ref[...])


def kernel(node_syms, W, b, out_W, out_b):
    syms = node_syms.astype(jnp.int32)
    leaf_b = _make_leaf_gather()(b, syms[_LEAVES - 1:])
    res = pl.pallas_call(
        _sweep_kernel,
        out_shape=jax.ShapeDtypeStruct((1, _OUT), jnp.float32),
        compiler_params=pltpu.CompilerParams(
            vmem_limit_bytes=100 * 1024 * 1024),
    )(syms.reshape(_N, 1),
      W.reshape(_A * _DIM, _DIM), b, leaf_b, out_W, out_b.reshape(1, _OUT))
    return res[0]


# speedup vs baseline: 2.6458x; 2.6458x over previous
import jax
import jax.numpy as jnp
from jax.experimental import pallas as pl
from jax.experimental.pallas import tpu as pltpu

_D = 13
_N = 2 ** _D - 1
_DIM = 128
_A = 16
_OUT = 10

_CHUNK = 512


def _sweep_kernel(syms_ref, w_ref, b_ref, out_w_ref, out_b_ref, out_ref):
    bv = b_ref[...]
    w_half = (w_ref[...] * 0.5).astype(jnp.bfloat16)
    b_cols = jax.lax.broadcast_in_dim(
        jnp.transpose(bv), (_A, _DIM, _A), (1, 2)
    ).reshape(_A * _DIM, _A).astype(jnp.bfloat16)
    w_aug = jnp.concatenate([w_half, b_cols], axis=1)

    a_iota = jax.lax.broadcasted_iota(jnp.int32, (1, _A), 1)

    def sym_slice(level):
        start = (1 << level) - 1
        return syms_ref[start:start + (1 << level)]

    s_leaf = (sym_slice(_D - 1) == a_iota).astype(jnp.float32)
    prev = jnp.tanh(jnp.dot(s_leaf, bv, preferred_element_type=jnp.float32))

    for level in range(_D - 2, -1, -1):
        length = 1 << level
        c3 = prev.reshape(length, 2, _DIM)
        x2 = c3[:, 0, :] + c3[:, 1, :]
        syms = sym_slice(level)
        s_oh = (syms == a_iota).astype(jnp.bfloat16)
        xa = jnp.concatenate(
            [x2.astype(jnp.bfloat16), s_oh], axis=1)
        sb = jnp.broadcast_to(syms, (length, _DIM))
        chunks = []
        for c0 in range(0, length, _CHUNK):
            c1 = min(c0 + _CHUNK, length)
            y_all = jax.lax.dot_general(
                xa[c0:c1], w_aug, (((1,), (1,)), ((), ())),
                preferred_element_type=jnp.float32)
            sbc = sb[c0:c1]
            acc = jnp.where(sbc == 0, y_all[:, 0:_DIM], 0.0)
            for a in range(1, _A):
                acc = acc + jnp.where(
                    sbc == a, y_all[:, a * _DIM:(a + 1) * _DIM], 0.0)
            chunks.append(acc)
        y = chunks[0] if len(chunks) == 1 else jnp.concatenate(chunks, axis=0)
        prev = jnp.tanh(y)

    out_ref[...] = (
        jax.lax.dot_general(prev, out_w_ref[...], (((1,), (1,)), ((), ())),
                            preferred_element_type=jnp.float32)
        + out_b_ref[...])


def kernel(node_syms, W, b, out_W, out_b):
    res = pl.pallas_call(
        _sweep_kernel,
        out_shape=jax.ShapeDtypeStruct((1, _OUT), jnp.float32),
        compiler_params=pltpu.CompilerParams(
            vmem_limit_bytes=100 * 1024 * 1024),
    )(node_syms.astype(jnp.int32).reshape(_N, 1),
      W.reshape(_A * _DIM, _DIM), b, out_W, out_b.reshape(1, _OUT))
    return res[0]
